# baseline (speedup 1.0000x reference)
import functools

import jax
import jax.numpy as jnp
from jax import lax
from jax.experimental import pallas as pl
from jax.experimental.pallas import tpu as pltpu
from jax.experimental.pallas import tpu_sc as plsc

_COMMITMENT_COST = 0.25
_NT = 1024
_NW = 32
_CHUNK = 128


def _prep_body(cb_ref, cbm2t_ref, c2_ref, ktab_ref):
    cb = cb_ref[...]
    cbm2t_ref[...] = jnp.swapaxes((-2.0 * cb).astype(jnp.bfloat16), 0, 1)
    c2 = jnp.sum(cb * cb, axis=1, keepdims=True)
    c2_ref[...] = jnp.swapaxes(c2, 0, 1)
    kk = cb_ref.shape[0]
    kcol = jax.lax.broadcasted_iota(jnp.int32, (kk, 8), 0)
    ccol = jax.lax.broadcasted_iota(jnp.int32, (kk, 8), 1)
    hi = (kcol // 16).astype(jnp.float32)
    lo = (kcol % 16).astype(jnp.float32)
    ktab = jnp.where(ccol == 0, hi,
                     jnp.where(ccol == 1, lo,
                               jnp.where(ccol == 2, 1.0, 0.0)))
    ktab_ref[...] = ktab


def _vq_body(z_ref, cbm2t_ref, c2_ref, ktab_ref, idx_ref, ls_ref):
    k = cbm2t_ref.shape[1]
    nt = z_ref.shape[0]
    zt = z_ref[...]
    s_neg = jax.lax.dot_general(
        zt.astype(jnp.bfloat16), cbm2t_ref[...],
        (((1,), (0,)), ((), ())),
        preferred_element_type=jnp.float32)
    z2 = jnp.sum(zt * zt, axis=1, keepdims=True)
    dist = (z2 + s_neg) + c2_ref[...]
    mins = jnp.min(dist, axis=1, keepdims=True)
    mask = jnp.where(dist == mins, 1.0, 0.0)
    r = jax.lax.dot_general(mask, ktab_ref[...],
                            (((1,), (0,)), ((), ())),
                            preferred_element_type=jnp.float32)
    cnt = r[:, 2:3]

    @pl.when(jnp.max(cnt) < 1.5)
    def _fast():
        idxf = (r[:, 0:1] * 16.0 + r[:, 1:2]).astype(jnp.int32)
        idx_ref[0] = jnp.swapaxes(idxf, 0, 1)

    @pl.when(jnp.max(cnt) >= 1.5)
    def _tie():
        kiota = jax.lax.broadcasted_iota(jnp.int32, (nt, k), 1)
        idx = jnp.min(jnp.where(dist == mins, kiota, k),
                      axis=1, keepdims=True)
        idx_ref[0] = jnp.swapaxes(idx, 0, 1)

    @pl.when(pl.program_id(0) == 0)
    def _init():
        ls_ref[...] = jnp.zeros_like(ls_ref)

    ls_ref[...] += mins


def _sc_body(cb_ref, idx_ref, zq_ref, idx_v, rows_v, sem0, sem1, so0, so1,
             *, n, chunk):
    per_w = n // _NW
    nchunk = per_w // chunk
    wid = lax.axis_index("s") * 2 + lax.axis_index("c")
    base = wid * per_w
    pltpu.sync_copy(idx_ref.at[pl.ds(base, per_w)], idx_v)
    gsems = (sem0, sem1)
    osems = (so0, so1)
    out_h = [None, None]
    for c in range(nchunk):
        buf = c % 2
        if out_h[buf] is not None:
            out_h[buf].wait()
        pltpu.async_copy(
            cb_ref.at[idx_v.at[pl.ds(c * chunk, chunk)]],
            rows_v.at[buf], gsems[buf]).wait()
        out_h[buf] = pltpu.async_copy(
            rows_v.at[buf], zq_ref.at[pl.ds(base + c * chunk, chunk)],
            osems[buf])
    out_h[0].wait()
    out_h[1].wait()


def kernel(z, codebook):
    b, d, h, w = z.shape
    kk = codebook.shape[0]
    hw = h * w
    n = b * hw
    nt = min(_NT, n)
    grid = n // nt
    zr = z.transpose(0, 2, 3, 1).reshape(n, d)

    cbm2t, c2, ktab = pl.pallas_call(
        _prep_body,
        out_shape=[
            jax.ShapeDtypeStruct((d, kk), jnp.bfloat16),
            jax.ShapeDtypeStruct((1, kk), jnp.float32),
            jax.ShapeDtypeStruct((kk, 8), jnp.float32),
        ],
    )(codebook)

    idx, ls = pl.pallas_call(
        _vq_body,
        grid=(grid,),
        in_specs=[
            pl.BlockSpec((nt, d), lambda i: (i, 0)),
            pl.BlockSpec((d, kk), lambda i: (0, 0)),
            pl.BlockSpec((1, kk), lambda i: (0, 0)),
            pl.BlockSpec((kk, 8), lambda i: (0, 0)),
        ],
        out_specs=[
            pl.BlockSpec((1, 1, nt), lambda i: (i, 0, 0)),
            pl.BlockSpec((nt, 1), lambda i: (0, 0)),
        ],
        out_shape=[
            jax.ShapeDtypeStruct((grid, 1, nt), jnp.int32),
            jax.ShapeDtypeStruct((nt, 1), jnp.float32),
        ],
    )(zr, cbm2t, c2, ktab)

    sc = pl.kernel(
        functools.partial(_sc_body, n=n, chunk=_CHUNK),
        out_type=jax.ShapeDtypeStruct((n, d), jnp.float32),
        mesh=plsc.VectorSubcoreMesh(core_axis_name="c", subcore_axis_name="s"),
        compiler_params=pltpu.CompilerParams(needs_layout_passes=False),
        scratch_types=[
            pltpu.VMEM((n // _NW,), jnp.int32),
            pltpu.VMEM((2, _CHUNK, d), jnp.float32),
            pltpu.SemaphoreType.DMA,
            pltpu.SemaphoreType.DMA,
            pltpu.SemaphoreType.DMA,
            pltpu.SemaphoreType.DMA,
        ],
    )
    zq = sc(codebook, idx.reshape(-1))

    zq_out = zq.reshape(b, h, w, d).transpose(0, 3, 1, 2)
    idx_out = idx.reshape(b, h, w)
    mse = jnp.sum(ls) / (b * d * hw)
    vq_loss = mse + _COMMITMENT_COST * mse
    return zq_out, idx_out, vq_loss
